# baseline (speedup 1.0000x reference)
import functools

import jax
import jax.numpy as jnp
from jax import lax
from jax.experimental import pallas as pl
from jax.experimental.pallas import tpu as pltpu
from jax.experimental.pallas import tpu_sc as plsc

D = 1024
O = 1024
E = 8
TOPK = 2
N = 4096
A = N * TOPK
BLK = 256
G = A // BLK + E
PADROWS = G * BLK
RB = 512
R, C = 32, 256

NC, NS = 2, 16
NW = NC * NS
APW = A // NW
CH = 64
NCH = APW // CH


def _router_body(x_ref, gw_ref, gb_ref, e_ref, p_ref):
    x = x_ref[...]
    gw = gw_ref[...]
    logits = lax.dot_general(
        x, gw, (((1,), (1,)), ((), ())),
        preferred_element_type=jnp.float32)
    logits = logits + gb_ref[...]
    iota = lax.broadcasted_iota(jnp.int32, (RB, E), 1)
    m = jnp.max(logits, axis=1, keepdims=True)
    ex = jnp.exp(logits - m)
    probs = ex / jnp.sum(ex, axis=1, keepdims=True)
    is1 = logits == m
    i1 = jnp.min(jnp.where(is1, iota, E), axis=1, keepdims=True)
    oh1 = iota == i1
    p1 = jnp.sum(jnp.where(oh1, probs, 0.0), axis=1, keepdims=True)
    l2 = jnp.where(oh1, -jnp.inf, logits)
    m2 = jnp.max(l2, axis=1, keepdims=True)
    i2 = jnp.min(jnp.where(l2 == m2, iota, E), axis=1, keepdims=True)
    oh2 = iota == i2
    p2 = jnp.sum(jnp.where(oh2, probs, 0.0), axis=1, keepdims=True)
    e_ref[...] = jnp.concatenate([i1, i2], axis=1)
    p_ref[...] = jnp.concatenate([p1, p2], axis=1)


def _router(xf, gate_W, gate_b):
    nblk = N // RB
    return pl.pallas_call(
        _router_body,
        grid=(nblk,),
        in_specs=[
            pl.BlockSpec((RB, D), lambda i: (i, 0)),
            pl.BlockSpec((E, D), lambda i: (0, 0)),
            pl.BlockSpec((1, E), lambda i: (0, 0)),
        ],
        out_specs=[
            pl.BlockSpec((RB, TOPK), lambda i: (i, 0)),
            pl.BlockSpec((RB, TOPK), lambda i: (i, 0)),
        ],
        out_shape=[
            jax.ShapeDtypeStruct((N, TOPK), jnp.int32),
            jax.ShapeDtypeStruct((N, TOPK), jnp.float32),
        ],
    )(xf, gate_W, gate_b.reshape(1, E))


def _meta_body(e_ref, pos_ref, bexp_ref):
    hi = lax.Precision.HIGHEST
    ev = e_ref[...]
    cu = lax.broadcasted_iota(jnp.int32, (C, C), 0)
    cv = lax.broadcasted_iota(jnp.int32, (C, C), 1)
    tri_c = (cu < cv).astype(jnp.float32)
    ru = lax.broadcasted_iota(jnp.int32, (R, R), 0)
    rv = lax.broadcasted_iota(jnp.int32, (R, R), 1)
    tri_r = (rv < ru).astype(jnp.float32)
    pos = jnp.zeros((R, C), jnp.float32)
    start = jnp.zeros((1, 1), jnp.float32)
    starts = []
    for e in range(E):
        m = (ev == e).astype(jnp.float32)
        p_in = lax.dot_general(m, tri_c, (((1,), (0,)), ((), ())),
                               preferred_element_type=jnp.float32, precision=hi)
        t = jnp.sum(m, axis=1, keepdims=True)
        p_row = lax.dot_general(tri_r, t, (((1,), (0,)), ((), ())),
                                preferred_element_type=jnp.float32, precision=hi)
        rank = p_in + p_row
        cnt = jnp.sum(t, axis=0, keepdims=True)
        nb = jnp.floor((cnt + (BLK - 1)) * (1.0 / BLK))
        starts.append(start)
        pos = pos + m * (start * BLK + rank)
        start = start + nb
    pos_ref[...] = jnp.clip(pos, 0, PADROWS - 1).astype(jnp.int32)
    gi = lax.broadcasted_iota(jnp.int32, (1, 64), 1).astype(jnp.float32)
    be = jnp.zeros((1, 64), jnp.float32)
    for e in range(E):
        be = be + (starts[e] <= gi).astype(jnp.float32)
    bexp = jnp.clip(be - 1.0, 0, E - 1)
    gremap = jnp.minimum(gi, jnp.maximum(start - 1.0, 0.0))
    bexp_ref[...] = jnp.concatenate([bexp, gremap], axis=0).astype(jnp.int32)


def _meta(e2):
    return pl.pallas_call(
        _meta_body,
        out_shape=[
            jax.ShapeDtypeStruct((R, C), jnp.int32),
            jax.ShapeDtypeStruct((2, 64), jnp.int32),
        ],
    )(e2)


def _sc_mesh():
    return plsc.VectorSubcoreMesh(
        core_axis_name="c", subcore_axis_name="s", num_cores=NC, num_subcores=NS)


TPW = N // NW
DCH = 32
DNCH = TPW // DCH
UCH = 32
UNCH = APW // UCH


def _dispatch(xf, pos0, pos1):
    def body(x_hbm, p0_hbm, p1_hbm, xs_hbm,
             xb0, xb1, i00, i01, i10, i11, sem_g, sem_s):
        wid = lax.axis_index("s") * NC + lax.axis_index("c")
        t0 = wid * TPW
        xbufs = [xb0, xb1]
        p0b, p1b = [i00, i01], [i10, i11]
        g = pltpu.async_copy(x_hbm.at[pl.ds(t0, DCH)], xb0, sem_g)
        prev_s = []
        for ch in range(DNCH):
            par = ch % 2
            off = t0 + ch * DCH
            pltpu.sync_copy(p0_hbm.at[pl.ds(off, DCH)], p0b[par])
            pltpu.sync_copy(p1_hbm.at[pl.ds(off, DCH)], p1b[par])
            g.wait()
            s0 = pltpu.async_copy(xbufs[par], xs_hbm.at[p0b[par]], sem_s)
            s1 = pltpu.async_copy(xbufs[par], xs_hbm.at[p1b[par]], sem_s)
            for s in prev_s:
                s.wait()
            if ch + 1 < DNCH:
                g = pltpu.async_copy(
                    x_hbm.at[pl.ds(off + DCH, DCH)], xbufs[1 - par], sem_g)
            prev_s = [s0, s1]
        for s in prev_s:
            s.wait()

    return pl.kernel(
        body,
        out_type=jax.ShapeDtypeStruct((PADROWS, D), jnp.float32),
        mesh=_sc_mesh(),
        scratch_types=[
            pltpu.VMEM((DCH, D), jnp.float32),
            pltpu.VMEM((DCH, D), jnp.float32),
            pltpu.VMEM((DCH,), jnp.int32),
            pltpu.VMEM((DCH,), jnp.int32),
            pltpu.VMEM((DCH,), jnp.int32),
            pltpu.VMEM((DCH,), jnp.int32),
            pltpu.SemaphoreType.DMA,
            pltpu.SemaphoreType.DMA,
        ],
    )(xf, pos0, pos1)


def _unsort(ys, posk):
    def body(ys_hbm, pk_hbm, ysu_hbm, ya, yb, ia, ib, sem_g, sem_w):
        wid = lax.axis_index("s") * NC + lax.axis_index("c")
        t0 = wid * APW
        yv, iv = [ya, yb], [ia, ib]
        pltpu.sync_copy(pk_hbm.at[pl.ds(t0, UCH)], ia)
        g = pltpu.async_copy(ys_hbm.at[ia], ya, sem_g)
        wr_prev = None
        for ch in range(UNCH):
            par = ch % 2
            off = t0 + ch * UCH
            g.wait()
            if ch + 1 < UNCH:
                pltpu.sync_copy(pk_hbm.at[pl.ds(off + UCH, UCH)], iv[1 - par])
                if wr_prev is not None:
                    wr_prev.wait()
                    wr_prev = None
                g = pltpu.async_copy(ys_hbm.at[iv[1 - par]], yv[1 - par], sem_g)
            if wr_prev is not None:
                wr_prev.wait()
            wr_prev = pltpu.async_copy(yv[par], ysu_hbm.at[pl.ds(off, UCH)], sem_w)
        wr_prev.wait()

    return pl.kernel(
        body,
        out_type=jax.ShapeDtypeStruct((A, O), jnp.float32),
        mesh=_sc_mesh(),
        scratch_types=[
            pltpu.VMEM((UCH, O), jnp.float32),
            pltpu.VMEM((UCH, O), jnp.float32),
            pltpu.VMEM((UCH,), jnp.int32),
            pltpu.VMEM((UCH,), jnp.int32),
            pltpu.SemaphoreType.DMA,
            pltpu.SemaphoreType.DMA,
        ],
    )(ys, posk)


def _comb_body(y0_ref, y1_ref, w_ref, out_ref):
    w = w_ref[...]
    out_ref[...] = y0_ref[...] * w[:, 0:1] + y1_ref[...] * w[:, 1:2]


def _combine(y0, y1, sel_p):
    nblk = N // RB
    return pl.pallas_call(
        _comb_body,
        grid=(nblk,),
        in_specs=[
            pl.BlockSpec((RB, O), lambda i: (i, 0)),
            pl.BlockSpec((RB, O), lambda i: (i, 0)),
            pl.BlockSpec((RB, TOPK), lambda i: (i, 0)),
        ],
        out_specs=pl.BlockSpec((RB, O), lambda i: (i, 0)),
        out_shape=jax.ShapeDtypeStruct((N, O), jnp.float32),
    )(y0, y1, sel_p)


def _gmm_body(be_ref, xs_ref, w_ref, b_ref, out_ref):
    acc = lax.dot_general(
        xs_ref[...], w_ref[0], (((1,), (1,)), ((), ())),
        preferred_element_type=jnp.float32)
    out_ref[...] = acc + b_ref[0]


def _gmm(bexp, xs, expert_W, expert_b):
    grid_spec = pltpu.PrefetchScalarGridSpec(
        num_scalar_prefetch=1,
        grid=(G,),
        in_specs=[
            pl.BlockSpec((BLK, D), lambda g, be: (be[1, g], 0)),
            pl.BlockSpec((1, O, D), lambda g, be: (be[0, g], 0, 0)),
            pl.BlockSpec((1, 1, O), lambda g, be: (be[0, g], 0, 0)),
        ],
        out_specs=pl.BlockSpec((BLK, O), lambda g, be: (be[1, g], 0)),
    )
    return pl.pallas_call(
        _gmm_body,
        grid_spec=grid_spec,
        out_shape=jax.ShapeDtypeStruct((PADROWS, O), jnp.float32),
    )(bexp, xs, expert_W, expert_b.reshape(E, 1, O))


def kernel(x, gate_W, gate_b, expert_W, expert_b):
    B, S, _ = x.shape
    xf = x.reshape(N, D)
    sel_e, sel_p = _router(xf, gate_W, gate_b)
    pos, bexp = _meta(sel_e.reshape(R, C))
    pos2 = pos.reshape(N, TOPK)
    pos0, pos1 = pos2[:, 0], pos2[:, 1]
    xs = _dispatch(xf, pos0, pos1)
    ys = _gmm(bexp, xs, expert_W, expert_b)
    posk = jnp.concatenate([pos0, pos1])
    ysu = _unsort(ys, posk)
    out = _combine(ysu[:N], ysu[N:], sel_p)
    return out.reshape(B, S, O)


# speedup vs baseline: 1.0333x; 1.0333x over previous
import functools

import jax
import jax.numpy as jnp
from jax import lax
from jax.experimental import pallas as pl
from jax.experimental.pallas import tpu as pltpu
from jax.experimental.pallas import tpu_sc as plsc

D = 1024
O = 1024
E = 8
TOPK = 2
N = 4096
A = N * TOPK
BLK = 256
G = A // BLK + E
PADROWS = G * BLK
RB = 512
R, C = 32, 256

NC, NS = 2, 16
NW = NC * NS
APW = A // NW
CH = 64
NCH = APW // CH


def _router_body(x_ref, gw_ref, gb_ref, e_ref, p_ref):
    x = x_ref[...]
    gw = gw_ref[...]
    logits = lax.dot_general(
        x, gw, (((1,), (1,)), ((), ())),
        preferred_element_type=jnp.float32)
    logits = logits + gb_ref[...]
    iota = lax.broadcasted_iota(jnp.int32, (RB, E), 1)
    m = jnp.max(logits, axis=1, keepdims=True)
    ex = jnp.exp(logits - m)
    probs = ex / jnp.sum(ex, axis=1, keepdims=True)
    is1 = logits == m
    i1 = jnp.min(jnp.where(is1, iota, E), axis=1, keepdims=True)
    oh1 = iota == i1
    p1 = jnp.sum(jnp.where(oh1, probs, 0.0), axis=1, keepdims=True)
    l2 = jnp.where(oh1, -jnp.inf, logits)
    m2 = jnp.max(l2, axis=1, keepdims=True)
    i2 = jnp.min(jnp.where(l2 == m2, iota, E), axis=1, keepdims=True)
    oh2 = iota == i2
    p2 = jnp.sum(jnp.where(oh2, probs, 0.0), axis=1, keepdims=True)
    e_ref[...] = jnp.concatenate([i1, i2], axis=1)
    p_ref[...] = jnp.concatenate([p1, p2], axis=1)


def _router(xf, gate_W, gate_b):
    nblk = N // RB
    return pl.pallas_call(
        _router_body,
        grid=(nblk,),
        in_specs=[
            pl.BlockSpec((RB, D), lambda i: (i, 0)),
            pl.BlockSpec((E, D), lambda i: (0, 0)),
            pl.BlockSpec((1, E), lambda i: (0, 0)),
        ],
        out_specs=[
            pl.BlockSpec((RB, TOPK), lambda i: (i, 0)),
            pl.BlockSpec((RB, TOPK), lambda i: (i, 0)),
        ],
        out_shape=[
            jax.ShapeDtypeStruct((N, TOPK), jnp.int32),
            jax.ShapeDtypeStruct((N, TOPK), jnp.float32),
        ],
    )(xf, gate_W, gate_b.reshape(1, E))


def _meta_body(e_ref, pos_ref, bexp_ref):
    hi = lax.Precision.HIGHEST
    ev = e_ref[...]
    cu = lax.broadcasted_iota(jnp.int32, (C, C), 0)
    cv = lax.broadcasted_iota(jnp.int32, (C, C), 1)
    tri_c = (cu < cv).astype(jnp.float32)
    ru = lax.broadcasted_iota(jnp.int32, (R, R), 0)
    rv = lax.broadcasted_iota(jnp.int32, (R, R), 1)
    tri_r = (rv < ru).astype(jnp.float32)
    pos = jnp.zeros((R, C), jnp.float32)
    start = jnp.zeros((1, 1), jnp.float32)
    starts = []
    for e in range(E):
        m = (ev == e).astype(jnp.float32)
        p_in = lax.dot_general(m, tri_c, (((1,), (0,)), ((), ())),
                               preferred_element_type=jnp.float32, precision=hi)
        t = jnp.sum(m, axis=1, keepdims=True)
        p_row = lax.dot_general(tri_r, t, (((1,), (0,)), ((), ())),
                                preferred_element_type=jnp.float32, precision=hi)
        rank = p_in + p_row
        cnt = jnp.sum(t, axis=0, keepdims=True)
        nb = jnp.floor((cnt + (BLK - 1)) * (1.0 / BLK))
        starts.append(start)
        pos = pos + m * (start * BLK + rank)
        start = start + nb
    pos_ref[...] = jnp.clip(pos, 0, PADROWS - 1).astype(jnp.int32)
    gi = lax.broadcasted_iota(jnp.int32, (1, 64), 1).astype(jnp.float32)
    be = jnp.zeros((1, 64), jnp.float32)
    for e in range(E):
        be = be + (starts[e] <= gi).astype(jnp.float32)
    bexp = jnp.clip(be - 1.0, 0, E - 1)
    gremap = jnp.minimum(gi, jnp.maximum(start - 1.0, 0.0))
    bexp_ref[...] = jnp.concatenate([bexp, gremap], axis=0).astype(jnp.int32)


def _meta(e2):
    return pl.pallas_call(
        _meta_body,
        out_shape=[
            jax.ShapeDtypeStruct((R, C), jnp.int32),
            jax.ShapeDtypeStruct((2, 64), jnp.int32),
        ],
    )(e2)


def _sc_mesh():
    return plsc.VectorSubcoreMesh(
        core_axis_name="c", subcore_axis_name="s", num_cores=NC, num_subcores=NS)


TPW = N // NW
DCH = 32
DNCH = TPW // DCH
CCH = 16
CNCH = TPW // CCH


def _dispatch(xf, pos0, pos1):
    def body(x_hbm, p0_hbm, p1_hbm, xs_hbm,
             xb0, xb1, i00, i01, i10, i11, sem_g, sem_s):
        wid = lax.axis_index("s") * NC + lax.axis_index("c")
        t0 = wid * TPW
        xbufs = [xb0, xb1]
        p0b, p1b = [i00, i01], [i10, i11]
        g = pltpu.async_copy(x_hbm.at[pl.ds(t0, DCH)], xb0, sem_g)
        prev_s = []
        for ch in range(DNCH):
            par = ch % 2
            off = t0 + ch * DCH
            pltpu.sync_copy(p0_hbm.at[pl.ds(off, DCH)], p0b[par])
            pltpu.sync_copy(p1_hbm.at[pl.ds(off, DCH)], p1b[par])
            g.wait()
            s0 = pltpu.async_copy(xbufs[par], xs_hbm.at[p0b[par]], sem_s)
            s1 = pltpu.async_copy(xbufs[par], xs_hbm.at[p1b[par]], sem_s)
            for s in prev_s:
                s.wait()
            if ch + 1 < DNCH:
                g = pltpu.async_copy(
                    x_hbm.at[pl.ds(off + DCH, DCH)], xbufs[1 - par], sem_g)
            prev_s = [s0, s1]
        for s in prev_s:
            s.wait()

    return pl.kernel(
        body,
        out_type=jax.ShapeDtypeStruct((PADROWS, D), jnp.float32),
        mesh=_sc_mesh(),
        scratch_types=[
            pltpu.VMEM((DCH, D), jnp.float32),
            pltpu.VMEM((DCH, D), jnp.float32),
            pltpu.VMEM((DCH,), jnp.int32),
            pltpu.VMEM((DCH,), jnp.int32),
            pltpu.VMEM((DCH,), jnp.int32),
            pltpu.VMEM((DCH,), jnp.int32),
            pltpu.SemaphoreType.DMA,
            pltpu.SemaphoreType.DMA,
        ],
    )(xf, pos0, pos1)


def _unsort_combine(ys, pos0, pos1, w0, w1):
    def weighted_add(y0r, y1r, w0v, w1v):
        w0all = w0v[...]
        w1all = w1v[...]
        lane0 = lax.iota(jnp.int32, 16) * 0

        def row(r, _):
            idx = lane0 + r
            b0 = w0all.at[idx].get(mode="promise_in_bounds")
            b1 = w1all.at[idx].get(mode="promise_in_bounds")

            def col(c, _):
                sl = pl.ds(c * 16, 16)
                y0r[r, sl] = y0r[r, sl] * b0 + y1r[r, sl] * b1
                return 0

            lax.fori_loop(0, O // 16, col, 0, unroll=16)
            return 0

        lax.fori_loop(0, CCH, row, 0)

    def body(ys_hbm, p0_hbm, p1_hbm, w0_hbm, w1_hbm, out_hbm,
             y0a, y0b, y1a, y1b, i0a, i0b, i1a, i1b,
             wa0, wb0, wa1, wb1, sem_g, sem_w):
        wid = lax.axis_index("s") * NC + lax.axis_index("c")
        t0 = wid * TPW
        y0v, y1v = [y0a, y0b], [y1a, y1b]
        i0v, i1v = [i0a, i0b], [i1a, i1b]
        w0v, w1v = [wa0, wb0], [wa1, wb1]
        pltpu.sync_copy(p0_hbm.at[pl.ds(t0, CCH)], i0a)
        pltpu.sync_copy(p1_hbm.at[pl.ds(t0, CCH)], i1a)
        g0 = pltpu.async_copy(ys_hbm.at[i0a], y0a, sem_g)
        g1 = pltpu.async_copy(ys_hbm.at[i1a], y1a, sem_g)
        wr_prev = None
        for ch in range(CNCH):
            par = ch % 2
            off = t0 + ch * CCH
            pltpu.sync_copy(w0_hbm.at[pl.ds(off, CCH)], w0v[par])
            pltpu.sync_copy(w1_hbm.at[pl.ds(off, CCH)], w1v[par])
            g0.wait()
            g1.wait()
            if ch + 1 < CNCH:
                noff = off + CCH
                pltpu.sync_copy(p0_hbm.at[pl.ds(noff, CCH)], i0v[1 - par])
                pltpu.sync_copy(p1_hbm.at[pl.ds(noff, CCH)], i1v[1 - par])
                if wr_prev is not None:
                    wr_prev.wait()
                    wr_prev = None
                g0 = pltpu.async_copy(ys_hbm.at[i0v[1 - par]], y0v[1 - par], sem_g)
                g1 = pltpu.async_copy(ys_hbm.at[i1v[1 - par]], y1v[1 - par], sem_g)
            weighted_add(y0v[par], y1v[par], w0v[par], w1v[par])
            if wr_prev is not None:
                wr_prev.wait()
            wr_prev = pltpu.async_copy(y0v[par], out_hbm.at[pl.ds(off, CCH)], sem_w)
        wr_prev.wait()

    return pl.kernel(
        body,
        out_type=jax.ShapeDtypeStruct((N, O), jnp.float32),
        mesh=_sc_mesh(),
        scratch_types=[
            pltpu.VMEM((CCH, O), jnp.float32),
            pltpu.VMEM((CCH, O), jnp.float32),
            pltpu.VMEM((CCH, O), jnp.float32),
            pltpu.VMEM((CCH, O), jnp.float32),
            pltpu.VMEM((CCH,), jnp.int32),
            pltpu.VMEM((CCH,), jnp.int32),
            pltpu.VMEM((CCH,), jnp.int32),
            pltpu.VMEM((CCH,), jnp.int32),
            pltpu.VMEM((CCH,), jnp.float32),
            pltpu.VMEM((CCH,), jnp.float32),
            pltpu.VMEM((CCH,), jnp.float32),
            pltpu.VMEM((CCH,), jnp.float32),
            pltpu.SemaphoreType.DMA,
            pltpu.SemaphoreType.DMA,
        ],
    )(ys, pos0, pos1, w0, w1)


def _gmm_body(be_ref, xs_ref, w_ref, b_ref, out_ref):
    acc = lax.dot_general(
        xs_ref[...], w_ref[0], (((1,), (1,)), ((), ())),
        preferred_element_type=jnp.float32)
    out_ref[...] = acc + b_ref[0]


def _gmm(bexp, xs, expert_W, expert_b):
    grid_spec = pltpu.PrefetchScalarGridSpec(
        num_scalar_prefetch=1,
        grid=(G,),
        in_specs=[
            pl.BlockSpec((BLK, D), lambda g, be: (be[1, g], 0)),
            pl.BlockSpec((1, O, D), lambda g, be: (be[0, g], 0, 0)),
            pl.BlockSpec((1, 1, O), lambda g, be: (be[0, g], 0, 0)),
        ],
        out_specs=pl.BlockSpec((BLK, O), lambda g, be: (be[1, g], 0)),
    )
    return pl.pallas_call(
        _gmm_body,
        grid_spec=grid_spec,
        out_shape=jax.ShapeDtypeStruct((PADROWS, O), jnp.float32),
    )(bexp, xs, expert_W, expert_b.reshape(E, 1, O))


def kernel(x, gate_W, gate_b, expert_W, expert_b):
    B, S, _ = x.shape
    xf = x.reshape(N, D)
    sel_e, sel_p = _router(xf, gate_W, gate_b)
    pos, bexp = _meta(sel_e.reshape(R, C))
    pos2 = pos.reshape(N, TOPK)
    pos0, pos1 = pos2[:, 0], pos2[:, 1]
    xs = _dispatch(xf, pos0, pos1)
    ys = _gmm(bexp, xs, expert_W, expert_b)
    out = _unsort_combine(ys, pos0, pos1, sel_p[:, 0], sel_p[:, 1])
    return out.reshape(B, S, O)
